# Initial kernel scaffold; baseline (speedup 1.0000x reference)
import jax
import jax.numpy as jnp
from jax.experimental import pallas as pl


def kernel(node_pos, node_scalar, type_ids, edge_index, params):
    raise NotImplementedError("write your pallas kernel here")


# speedup vs baseline: 2.3024x; 2.3024x over previous
import functools

import jax
import jax.numpy as jnp
from jax import lax
from jax.experimental import pallas as pl
from jax.experimental.pallas import tpu as pltpu
from jax.experimental.pallas import tpu_sc as plsc

NN = 50000
EE = 800000
HH = 64
XP = 16
NC = 2
NS = 16
CH = 128
EW = 25088
EPAD = EW * NC * NS
ET = EPAD // NS
HALF = NN // 2
SLAB = 1568
SH = HALF + 8
ILAST = SH - (NS - 1) * SLAB
LAST = HALF - (NS - 1) * SLAB

BE = 2048
BN = 5000

_SC_PARAMS = pltpu.CompilerParams(use_tc_tiling_on_sc=False)


def _silu(x):
    return x * jax.nn.sigmoid(x)


def _wt(Wb):
    W, b = Wb
    return W.T, b.reshape(1, -1)


def _embed_body(t_ref, s_ref, te_ref, w0, b0, w1, b1, w2, b2, o_ref):
    ids = t_ref[...]
    oh = (ids == lax.broadcasted_iota(jnp.int32, (BN, 5), 1)).astype(jnp.float32)
    h = oh @ te_ref[...]
    a = _silu(s_ref[...] @ w0[...] + b0[...])
    a = _silu(a @ w1[...] + b1[...])
    o_ref[...] = h + a @ w2[...] + b2[...]


def _embed(tids2, nscal, te, ps):
    w0, b0 = _wt(ps[0])
    w1, b1 = _wt(ps[1])
    w2, b2 = _wt(ps[2])
    full = lambda s: pl.BlockSpec(s, lambda i: (0, 0))
    return pl.pallas_call(
        _embed_body,
        grid=(NN // BN,),
        in_specs=[
            pl.BlockSpec((BN, 1), lambda i: (i, 0)),
            pl.BlockSpec((BN, 5), lambda i: (i, 0)),
            full((5, HH)), full((5, HH)), full((1, HH)),
            full((HH, HH)), full((1, HH)), full((HH, HH)), full((1, HH)),
        ],
        out_specs=pl.BlockSpec((BN, HH), lambda i: (i, 0)),
        out_shape=jax.ShapeDtypeStruct((NN, HH), jnp.float32),
    )(tids2, nscal, te, w0, b0, w1, b1, w2, b2)


def _edge_body(hs, hd, xs, xd, w1s, w1d, w1e, b1, w2, b2, w3, b3,
               wc1, bc1, wc2, bc2, m_ref, w_ref):
    rel = xs[...] - xd[...]
    d2 = jnp.sum(rel * rel, axis=1, keepdims=True)
    a = _silu(hs[...] @ w1s[...] + hd[...] @ w1d[...] + d2 * w1e[...] + b1[...])
    a = _silu(a @ w2[...] + b2[...])
    m = a @ w3[...] + b3[...]
    c = _silu(m @ wc1[...] + bc1[...])
    coef = jnp.tanh(c @ wc2[...] + bc2[...])
    m_ref[...] = m
    w_ref[...] = rel * coef


def _edge_tc(hs, hd, xs, xd, lp):
    W1, b1 = lp['edge'][0]
    w1s, w1d, w1e = W1[:, :HH].T, W1[:, HH:2 * HH].T, W1[:, 2 * HH:].T
    w2, b2 = _wt(lp['edge'][1])
    w3, b3 = _wt(lp['edge'][2])
    wc1, bc1 = _wt(lp['coord'][0])
    wc2, bc2 = _wt(lp['coord'][1])
    full = lambda s: pl.BlockSpec(s, lambda i: (0, 0))
    return pl.pallas_call(
        _edge_body,
        grid=(EPAD // BE,),
        in_specs=[
            pl.BlockSpec((BE, HH), lambda i: (i, 0)),
            pl.BlockSpec((BE, HH), lambda i: (i, 0)),
            pl.BlockSpec((BE, XP), lambda i: (i, 0)),
            pl.BlockSpec((BE, XP), lambda i: (i, 0)),
            full((HH, HH)), full((HH, HH)), full((1, HH)), full((1, HH)),
            full((HH, HH)), full((1, HH)), full((HH, HH)), full((1, HH)),
            full((HH, HH)), full((1, HH)), full((HH, 1)), full((1, 1)),
        ],
        out_specs=[
            pl.BlockSpec((BE, HH), lambda i: (i, 0)),
            pl.BlockSpec((BE, XP), lambda i: (i, 0)),
        ],
        out_shape=[
            jax.ShapeDtypeStruct((EPAD, HH), jnp.float32),
            jax.ShapeDtypeStruct((EPAD, XP), jnp.float32),
        ],
    )(hs, hd, xs, xd, w1s, w1d, w1e.reshape(1, HH), b1.reshape(1, HH),
      w2, b2, w3, b3, wc1, bc1, wc2, bc2.reshape(1, 1))


def _node_body(h, ma_ref, xp, dx_ref, w1h, w1m, b1, w2, b2, w3, b3,
               ho_ref, xo_ref):
    a = _silu(h[...] @ w1h[...] + ma_ref[...] @ w1m[...] + b1[...])
    a = _silu(a @ w2[...] + b2[...])
    ho_ref[...] = h[...] + a @ w3[...] + b3[...]
    xo_ref[...] = xp[...] + dx_ref[...]


def _node_tc(h, ma, xp, dx, lp):
    W1, b1 = lp['node'][0]
    w1h, w1m = W1[:, :HH].T, W1[:, HH:].T
    w2, b2 = _wt(lp['node'][1])
    w3, b3 = _wt(lp['node'][2])
    full = lambda s: pl.BlockSpec(s, lambda i: (0, 0))
    return pl.pallas_call(
        _node_body,
        grid=(NN // BN,),
        in_specs=[
            pl.BlockSpec((BN, HH), lambda i: (i, 0)),
            pl.BlockSpec((BN, HH), lambda i: (i, 0)),
            pl.BlockSpec((BN, XP), lambda i: (i, 0)),
            pl.BlockSpec((BN, XP), lambda i: (i, 0)),
            full((HH, HH)), full((HH, HH)), full((1, HH)),
            full((HH, HH)), full((1, HH)), full((HH, HH)), full((1, HH)),
        ],
        out_specs=[
            pl.BlockSpec((BN, HH), lambda i: (i, 0)),
            pl.BlockSpec((BN, XP), lambda i: (i, 0)),
        ],
        out_shape=[
            jax.ShapeDtypeStruct((NN, HH), jnp.float32),
            jax.ShapeDtypeStruct((NN, XP), jnp.float32),
        ],
    )(h, ma, xp, dx, w1h, w1m, b1.reshape(1, HH), w2, b2, w3, b3)


def _sc_gather(h, xp, sidx, didx):
    mesh = plsc.VectorSubcoreMesh(core_axis_name="c", subcore_axis_name="s")

    @functools.partial(
        pl.kernel,
        out_type=(
            jax.ShapeDtypeStruct((EPAD, HH), jnp.float32),
            jax.ShapeDtypeStruct((EPAD, HH), jnp.float32),
            jax.ShapeDtypeStruct((EPAD, XP), jnp.float32),
            jax.ShapeDtypeStruct((EPAD, XP), jnp.float32),
        ),
        mesh=mesh,
        scratch_types=[
            pltpu.VMEM((EW,), jnp.int32),
            pltpu.VMEM((EW,), jnp.int32),
            pltpu.VMEM((CH, HH), jnp.float32),
            pltpu.VMEM((CH, HH), jnp.float32),
            pltpu.VMEM((CH, XP), jnp.float32),
            pltpu.VMEM((CH, XP), jnp.float32),
            pltpu.SemaphoreType.DMA,
        ],
        compiler_params=_SC_PARAMS,
    )
    def k(h_hbm, x_hbm, s_hbm, d_hbm, hs_hbm, hd_hbm, xs_hbm, xd_hbm,
          sv, dv, hsb, hdb, xsb, xdb, sem):
        wid = lax.axis_index("s") * NC + lax.axis_index("c")
        base = wid * EW
        pltpu.sync_copy(s_hbm.at[pl.ds(base, EW)], sv)
        pltpu.sync_copy(d_hbm.at[pl.ds(base, EW)], dv)

        def chunk(j, carry):
            off = j * CH
            c1 = pltpu.async_copy(h_hbm.at[sv.at[pl.ds(off, CH)]], hsb, sem)
            c2 = pltpu.async_copy(h_hbm.at[dv.at[pl.ds(off, CH)]], hdb, sem)
            c3 = pltpu.async_copy(x_hbm.at[sv.at[pl.ds(off, CH)]], xsb, sem)
            c4 = pltpu.async_copy(x_hbm.at[dv.at[pl.ds(off, CH)]], xdb, sem)
            c1.wait(); c2.wait(); c3.wait(); c4.wait()
            pltpu.sync_copy(hsb, hs_hbm.at[pl.ds(base + off, CH)])
            pltpu.sync_copy(hdb, hd_hbm.at[pl.ds(base + off, CH)])
            pltpu.sync_copy(xsb, xs_hbm.at[pl.ds(base + off, CH)])
            pltpu.sync_copy(xdb, xd_hbm.at[pl.ds(base + off, CH)])
            return carry

        lax.fori_loop(0, EW // CH, chunk, 0)

    return k(h, xp, sidx, didx)


def _sc_scatter(vals, sidx, zeros, width, after):
    mesh = plsc.VectorSubcoreMesh(core_axis_name="c", subcore_axis_name="s")

    @functools.partial(
        pl.kernel,
        out_type=jax.ShapeDtypeStruct((NN, width), jnp.float32),
        mesh=mesh,
        scratch_types=[
            pltpu.VMEM((CH,), jnp.int32),
            pltpu.VMEM((CH,), jnp.int32),
            pltpu.VMEM((CH, width), jnp.float32),
            pltpu.VMEM_SHARED((SH, width), jnp.float32),
            pltpu.SemaphoreType.DMA,
        ],
        compiler_params=_SC_PARAMS,
    )
    def k(v_hbm, s_hbm, z_hbm, a_hbm, o_hbm, srcb, idxb, vb, shacc, sem):
        del a_hbm
        c = lax.axis_index("c")
        s = lax.axis_index("s")
        nbase = c * HALF
        rb = s * SLAB

        @pl.when(s < NS - 1)
        def _():
            pltpu.sync_copy(z_hbm.at[pl.ds(rb, SLAB)], shacc.at[pl.ds(rb, SLAB)])

        @pl.when(s == NS - 1)
        def _():
            pltpu.sync_copy(z_hbm.at[pl.ds(rb, ILAST)], shacc.at[pl.ds(rb, ILAST)])

        plsc.subcore_barrier()
        ebase = s * ET

        def chunk(j, carry):
            off = ebase + j * CH
            cs = pltpu.async_copy(s_hbm.at[pl.ds(off, CH)], srcb, sem)
            cv = pltpu.async_copy(v_hbm.at[pl.ds(off, CH)], vb, sem)
            cs.wait()
            cv.wait()

            def ivec(v, cc):
                t = srcb[pl.ds(v * 16, 16)] - nbase
                ok = (t >= 0) & (t < HALF)
                idxb[pl.ds(v * 16, 16)] = jnp.where(ok, t, HALF)
                return cc

            lax.fori_loop(0, CH // 16, ivec, 0)
            pltpu.sync_copy(vb, shacc.at[idxb], add=True)
            return carry

        lax.fori_loop(0, ET // CH, chunk, 0)
        plsc.subcore_barrier()

        @pl.when(s < NS - 1)
        def _():
            pltpu.sync_copy(shacc.at[pl.ds(rb, SLAB)],
                            o_hbm.at[pl.ds(nbase + rb, SLAB)])

        @pl.when(s == NS - 1)
        def _():
            pltpu.sync_copy(shacc.at[pl.ds(rb, LAST)],
                            o_hbm.at[pl.ds(nbase + rb, LAST)])

    return k(vals, sidx, zeros, after)


def kernel(node_pos, node_scalar, type_ids, edge_index, params):
    pad = EPAD - EE
    src = edge_index[0].astype(jnp.int32)
    dst = edge_index[1].astype(jnp.int32)
    src_g = jnp.concatenate([src, jnp.zeros((pad,), jnp.int32)])
    dst_g = jnp.concatenate([dst, jnp.zeros((pad,), jnp.int32)])
    src_s = jnp.concatenate([src, jnp.full((pad,), NN, jnp.int32)])
    z64 = jnp.zeros((SH, HH), jnp.float32)
    z16 = jnp.zeros((SH, XP), jnp.float32)

    xp = jnp.concatenate(
        [node_pos, jnp.zeros((NN, XP - 3), jnp.float32)], axis=1)
    h = _embed(type_ids.astype(jnp.int32).reshape(NN, 1), node_scalar,
               params['type_embed'], params['scalar'])

    for lp in params['layers']:
        hs, hd, xs, xd = _sc_gather(h, xp, src_g, dst_g)
        m, w = _edge_tc(hs, hd, xs, xd, lp)
        ma = _sc_scatter(m, src_s, z64, HH, w[:8])
        dx = _sc_scatter(w, src_s, z16, XP, ma[:8])
        h, xp = _node_tc(h, ma, xp, dx, lp)

    return h


# speedup vs baseline: 2.4205x; 1.0513x over previous
import functools

import jax
import jax.numpy as jnp
from jax import lax
from jax.experimental import pallas as pl
from jax.experimental.pallas import tpu as pltpu
from jax.experimental.pallas import tpu_sc as plsc

NN = 50000
EE = 800000
HH = 64
XP = 16
NC = 2
NS = 16
CH = 128
EW = 25088
EPAD = EW * NC * NS
ET = EPAD // NS
HALF = NN // 2
SLAB = 1568
SH = HALF + 8
ILAST = SH - (NS - 1) * SLAB
LAST = HALF - (NS - 1) * SLAB

BE = 2048
BN = 5000

_SC_PARAMS = pltpu.CompilerParams(use_tc_tiling_on_sc=False)


def _silu(x):
    return x * jax.nn.sigmoid(x)


def _wt(Wb):
    W, b = Wb
    return W.T, b.reshape(1, -1)


def _embed_body(t_ref, s_ref, te_ref, w0, b0, w1, b1, w2, b2, o_ref):
    ids = t_ref[...]
    oh = (ids == lax.broadcasted_iota(jnp.int32, (BN, 5), 1)).astype(jnp.float32)
    h = oh @ te_ref[...]
    a = _silu(s_ref[...] @ w0[...] + b0[...])
    a = _silu(a @ w1[...] + b1[...])
    o_ref[...] = h + a @ w2[...] + b2[...]


def _embed(tids2, nscal, te, ps):
    w0, b0 = _wt(ps[0])
    w1, b1 = _wt(ps[1])
    w2, b2 = _wt(ps[2])
    full = lambda s: pl.BlockSpec(s, lambda i: (0, 0))
    return pl.pallas_call(
        _embed_body,
        grid=(NN // BN,),
        in_specs=[
            pl.BlockSpec((BN, 1), lambda i: (i, 0)),
            pl.BlockSpec((BN, 5), lambda i: (i, 0)),
            full((5, HH)), full((5, HH)), full((1, HH)),
            full((HH, HH)), full((1, HH)), full((HH, HH)), full((1, HH)),
        ],
        out_specs=pl.BlockSpec((BN, HH), lambda i: (i, 0)),
        out_shape=jax.ShapeDtypeStruct((NN, HH), jnp.float32),
    )(tids2, nscal, te, w0, b0, w1, b1, w2, b2)


def _edge_body(hs, hd, xs, xd, w1s, w1d, w1e, b1, w2, b2, w3, b3,
               wc1, bc1, wc2, bc2, m_ref, w_ref):
    rel = xs[...] - xd[...]
    d2 = jnp.sum(rel * rel, axis=1, keepdims=True)
    a = _silu(hs[...] @ w1s[...] + hd[...] @ w1d[...] + d2 * w1e[...] + b1[...])
    a = _silu(a @ w2[...] + b2[...])
    m = a @ w3[...] + b3[...]
    c = _silu(m @ wc1[...] + bc1[...])
    coef = jnp.tanh(c @ wc2[...] + bc2[...])
    m_ref[...] = m
    w_ref[...] = rel * coef


def _edge_tc(hs, hd, xs, xd, lp):
    W1, b1 = lp['edge'][0]
    w1s, w1d, w1e = W1[:, :HH].T, W1[:, HH:2 * HH].T, W1[:, 2 * HH:].T
    w2, b2 = _wt(lp['edge'][1])
    w3, b3 = _wt(lp['edge'][2])
    wc1, bc1 = _wt(lp['coord'][0])
    wc2, bc2 = _wt(lp['coord'][1])
    full = lambda s: pl.BlockSpec(s, lambda i: (0, 0))
    return pl.pallas_call(
        _edge_body,
        grid=(EPAD // BE,),
        in_specs=[
            pl.BlockSpec((BE, HH), lambda i: (i, 0)),
            pl.BlockSpec((BE, HH), lambda i: (i, 0)),
            pl.BlockSpec((BE, XP), lambda i: (i, 0)),
            pl.BlockSpec((BE, XP), lambda i: (i, 0)),
            full((HH, HH)), full((HH, HH)), full((1, HH)), full((1, HH)),
            full((HH, HH)), full((1, HH)), full((HH, HH)), full((1, HH)),
            full((HH, HH)), full((1, HH)), full((HH, 1)), full((1, 1)),
        ],
        out_specs=[
            pl.BlockSpec((BE, HH), lambda i: (i, 0)),
            pl.BlockSpec((BE, XP), lambda i: (i, 0)),
        ],
        out_shape=[
            jax.ShapeDtypeStruct((EPAD, HH), jnp.float32),
            jax.ShapeDtypeStruct((EPAD, XP), jnp.float32),
        ],
    )(hs, hd, xs, xd, w1s, w1d, w1e.reshape(1, HH), b1.reshape(1, HH),
      w2, b2, w3, b3, wc1, bc1, wc2, bc2.reshape(1, 1))


def _node_body(h, ma_ref, xp, dx_ref, w1h, w1m, b1, w2, b2, w3, b3,
               ho_ref, xo_ref):
    a = _silu(h[...] @ w1h[...] + ma_ref[...] @ w1m[...] + b1[...])
    a = _silu(a @ w2[...] + b2[...])
    ho_ref[...] = h[...] + a @ w3[...] + b3[...]
    xo_ref[...] = xp[...] + dx_ref[...]


def _node_tc(h, ma, xp, dx, lp):
    W1, b1 = lp['node'][0]
    w1h, w1m = W1[:, :HH].T, W1[:, HH:].T
    w2, b2 = _wt(lp['node'][1])
    w3, b3 = _wt(lp['node'][2])
    full = lambda s: pl.BlockSpec(s, lambda i: (0, 0))
    return pl.pallas_call(
        _node_body,
        grid=(NN // BN,),
        in_specs=[
            pl.BlockSpec((BN, HH), lambda i: (i, 0)),
            pl.BlockSpec((BN, HH), lambda i: (i, 0)),
            pl.BlockSpec((BN, XP), lambda i: (i, 0)),
            pl.BlockSpec((BN, XP), lambda i: (i, 0)),
            full((HH, HH)), full((HH, HH)), full((1, HH)),
            full((HH, HH)), full((1, HH)), full((HH, HH)), full((1, HH)),
        ],
        out_specs=[
            pl.BlockSpec((BN, HH), lambda i: (i, 0)),
            pl.BlockSpec((BN, XP), lambda i: (i, 0)),
        ],
        out_shape=[
            jax.ShapeDtypeStruct((NN, HH), jnp.float32),
            jax.ShapeDtypeStruct((NN, XP), jnp.float32),
        ],
    )(h, ma, xp, dx, w1h, w1m, b1.reshape(1, HH), w2, b2, w3, b3)


def _sc_gather(h, xp, sidx, didx):
    mesh = plsc.VectorSubcoreMesh(core_axis_name="c", subcore_axis_name="s")

    @functools.partial(
        pl.kernel,
        out_type=(
            jax.ShapeDtypeStruct((EPAD, HH), jnp.float32),
            jax.ShapeDtypeStruct((EPAD, HH), jnp.float32),
            jax.ShapeDtypeStruct((EPAD, XP), jnp.float32),
            jax.ShapeDtypeStruct((EPAD, XP), jnp.float32),
        ),
        mesh=mesh,
        scratch_types=[
            pltpu.VMEM((EW,), jnp.int32),
            pltpu.VMEM((EW,), jnp.int32),
            [pltpu.VMEM((CH, HH), jnp.float32) for _ in range(2)],
            [pltpu.VMEM((CH, HH), jnp.float32) for _ in range(2)],
            [pltpu.VMEM((CH, XP), jnp.float32) for _ in range(2)],
            [pltpu.VMEM((CH, XP), jnp.float32) for _ in range(2)],
            [pltpu.SemaphoreType.DMA for _ in range(2)],
            [pltpu.SemaphoreType.DMA for _ in range(2)],
        ],
        compiler_params=_SC_PARAMS,
    )
    def k(h_hbm, x_hbm, s_hbm, d_hbm, hs_hbm, hd_hbm, xs_hbm, xd_hbm,
          sv, dv, hsb, hdb, xsb, xdb, sg, sw):
        wid = lax.axis_index("s") * NC + lax.axis_index("c")
        base = wid * EW
        pltpu.sync_copy(s_hbm.at[pl.ds(base, EW)], sv)
        pltpu.sync_copy(d_hbm.at[pl.ds(base, EW)], dv)

        def fire_g(off, b):
            pltpu.async_copy(h_hbm.at[sv.at[pl.ds(off, CH)]], hsb[b], sg[b])
            pltpu.async_copy(h_hbm.at[dv.at[pl.ds(off, CH)]], hdb[b], sg[b])
            pltpu.async_copy(x_hbm.at[sv.at[pl.ds(off, CH)]], xsb[b], sg[b])
            pltpu.async_copy(x_hbm.at[dv.at[pl.ds(off, CH)]], xdb[b], sg[b])

        def wait_g(b):
            pltpu.make_async_copy(h_hbm.at[pl.ds(0, CH)], hsb[b], sg[b]).wait()
            pltpu.make_async_copy(h_hbm.at[pl.ds(0, CH)], hdb[b], sg[b]).wait()
            pltpu.make_async_copy(x_hbm.at[pl.ds(0, CH)], xsb[b], sg[b]).wait()
            pltpu.make_async_copy(x_hbm.at[pl.ds(0, CH)], xdb[b], sg[b]).wait()

        def fire_w(off, b):
            pltpu.async_copy(hsb[b], hs_hbm.at[pl.ds(base + off, CH)], sw[b])
            pltpu.async_copy(hdb[b], hd_hbm.at[pl.ds(base + off, CH)], sw[b])
            pltpu.async_copy(xsb[b], xs_hbm.at[pl.ds(base + off, CH)], sw[b])
            pltpu.async_copy(xdb[b], xd_hbm.at[pl.ds(base + off, CH)], sw[b])

        def wait_w(b):
            pltpu.make_async_copy(hsb[b], hs_hbm.at[pl.ds(0, CH)], sw[b]).wait()
            pltpu.make_async_copy(hdb[b], hd_hbm.at[pl.ds(0, CH)], sw[b]).wait()
            pltpu.make_async_copy(xsb[b], xs_hbm.at[pl.ds(0, CH)], sw[b]).wait()
            pltpu.make_async_copy(xdb[b], xd_hbm.at[pl.ds(0, CH)], sw[b]).wait()

        fire_g(0, 0)
        fire_g(CH, 1)

        def step(t, carry):
            off = 2 * t * CH
            wait_g(0)
            fire_w(off, 0)
            wait_g(1)
            fire_w(off + CH, 1)
            wait_w(0)

            @pl.when(t < EW // CH // 2 - 1)
            def _():
                fire_g(off + 2 * CH, 0)

            wait_w(1)

            @pl.when(t < EW // CH // 2 - 1)
            def _():
                fire_g(off + 3 * CH, 1)

            return carry

        lax.fori_loop(0, EW // CH // 2, step, 0)

    return k(h, xp, sidx, didx)


def _sc_scatter(vals, sidx, zeros, width, after):
    mesh = plsc.VectorSubcoreMesh(core_axis_name="c", subcore_axis_name="s")

    @functools.partial(
        pl.kernel,
        out_type=jax.ShapeDtypeStruct((NN, width), jnp.float32),
        mesh=mesh,
        scratch_types=[
            [pltpu.VMEM((CH,), jnp.int32) for _ in range(2)],
            [pltpu.VMEM((CH,), jnp.int32) for _ in range(2)],
            [pltpu.VMEM((CH, width), jnp.float32) for _ in range(2)],
            pltpu.VMEM_SHARED((SH, width), jnp.float32),
            [pltpu.SemaphoreType.DMA for _ in range(2)],
        ],
        compiler_params=_SC_PARAMS,
    )
    def k(v_hbm, s_hbm, z_hbm, a_hbm, o_hbm, srcb, idxb, vb, shacc, sr):
        del a_hbm
        c = lax.axis_index("c")
        s = lax.axis_index("s")
        nbase = c * HALF
        rb = s * SLAB

        @pl.when(s < NS - 1)
        def _():
            pltpu.sync_copy(z_hbm.at[pl.ds(rb, SLAB)], shacc.at[pl.ds(rb, SLAB)])

        @pl.when(s == NS - 1)
        def _():
            pltpu.sync_copy(z_hbm.at[pl.ds(rb, ILAST)], shacc.at[pl.ds(rb, ILAST)])

        plsc.subcore_barrier()
        ebase = s * ET

        def fire_r(off, b):
            pltpu.async_copy(s_hbm.at[pl.ds(off, CH)], srcb[b], sr[b])
            pltpu.async_copy(v_hbm.at[pl.ds(off, CH)], vb[b], sr[b])

        def wait_r(b):
            pltpu.make_async_copy(s_hbm.at[pl.ds(0, CH)], srcb[b], sr[b]).wait()
            pltpu.make_async_copy(v_hbm.at[pl.ds(0, CH)], vb[b], sr[b]).wait()

        def half(t, b):
            off = ebase + (2 * t + b) * CH
            wait_r(b)

            def ivec(v, cc):
                tt = srcb[b][pl.ds(v * 16, 16)] - nbase
                ok = (tt >= 0) & (tt < HALF)
                idxb[b][pl.ds(v * 16, 16)] = jnp.where(ok, tt, HALF)
                return cc

            lax.fori_loop(0, CH // 16, ivec, 0)
            pltpu.sync_copy(vb[b], shacc.at[idxb[b]], add=True)

            @pl.when(t < ET // CH // 2 - 1)
            def _():
                fire_r(off + 2 * CH, b)

        fire_r(ebase, 0)
        fire_r(ebase + CH, 1)

        def step(t, carry):
            half(t, 0)
            half(t, 1)
            return carry

        lax.fori_loop(0, ET // CH // 2, step, 0)
        plsc.subcore_barrier()

        @pl.when(s < NS - 1)
        def _():
            pltpu.sync_copy(shacc.at[pl.ds(rb, SLAB)],
                            o_hbm.at[pl.ds(nbase + rb, SLAB)])

        @pl.when(s == NS - 1)
        def _():
            pltpu.sync_copy(shacc.at[pl.ds(rb, LAST)],
                            o_hbm.at[pl.ds(nbase + rb, LAST)])

    return k(vals, sidx, zeros, after)


def kernel(node_pos, node_scalar, type_ids, edge_index, params):
    pad = EPAD - EE
    src = edge_index[0].astype(jnp.int32)
    dst = edge_index[1].astype(jnp.int32)
    src_g = jnp.concatenate([src, jnp.zeros((pad,), jnp.int32)])
    dst_g = jnp.concatenate([dst, jnp.zeros((pad,), jnp.int32)])
    src_s = jnp.concatenate([src, jnp.full((pad,), NN, jnp.int32)])
    z64 = jnp.zeros((SH, HH), jnp.float32)
    z16 = jnp.zeros((SH, XP), jnp.float32)

    xp = jnp.concatenate(
        [node_pos, jnp.zeros((NN, XP - 3), jnp.float32)], axis=1)
    h = _embed(type_ids.astype(jnp.int32).reshape(NN, 1), node_scalar,
               params['type_embed'], params['scalar'])

    for lp in params['layers']:
        hs, hd, xs, xd = _sc_gather(h, xp, src_g, dst_g)
        m, w = _edge_tc(hs, hd, xs, xd, lp)
        ma = _sc_scatter(m, src_s, z64, HH, w[:8])
        dx = _sc_scatter(w, src_s, z16, XP, ma[:8])
        h, xp = _node_tc(h, ma, xp, dx, lp)

    return h
